# baseline (speedup 1.0000x reference)
import functools

import jax
import jax.numpy as jnp
from jax import lax
from jax.experimental import pallas as pl
from jax.experimental.pallas import tpu as pltpu
from jax.experimental.pallas import tpu_sc as plsc

_BATCH = 8192
_WIDTH = 1024
_TREES = 512
_DEPTH = 6
_NODES = 2 ** _DEPTH - 1
_LEAF0 = 2 ** (_DEPTH - 1) - 1
_WORDS = _WIDTH // 16

_NWORKERS = 32
_ROWS_PER_W = _BATCH // _NWORKERS
_SBLK = 32
_TVECS = _TREES // 16


def _pack_body(x_ref, p_ref, o_ref):
    acc = jnp.dot(x_ref[...], p_ref[...], preferred_element_type=jnp.float32)
    o_ref[...] = acc.astype(jnp.int32)


def _pack_bits(x):
    j = jnp.arange(2 * _WIDTH, dtype=jnp.int32)
    c = jnp.arange(2 * _WORDS, dtype=jnp.int32)
    tgt = (j % _WIDTH) // 16 + _WORDS * (j // _WIDTH)
    p = jnp.where(tgt[:, None] == c[None, :],
                  (2.0 ** (j % 16).astype(jnp.float32))[:, None],
                  0.0).astype(jnp.float32)
    xr = x.reshape(_BATCH // 2, 2 * _WIDTH)
    return pl.pallas_call(
        _pack_body,
        grid=(16,),
        in_specs=[
            pl.BlockSpec((_BATCH // 32, 2 * _WIDTH), lambda i: (i, 0)),
            pl.BlockSpec((2 * _WIDTH, 2 * _WORDS), lambda i: (0, 0)),
        ],
        out_specs=pl.BlockSpec((_BATCH // 32, 2 * _WORDS), lambda i: (i, 0)),
        out_shape=jax.ShapeDtypeStruct((_BATCH // 2, 2 * _WORDS), jnp.int32),
    )(xr, p)


@functools.partial(
    pl.kernel,
    mesh=plsc.VectorSubcoreMesh(core_axis_name="c", subcore_axis_name="s"),
    out_type=jax.ShapeDtypeStruct((_BATCH, _TREES), jnp.float32),
    compiler_params=pltpu.CompilerParams(needs_layout_passes=False),
    scratch_types=[
        pltpu.VMEM((_ROWS_PER_W * _WORDS,), jnp.int32),
        pltpu.VMEM((_TREES * _NODES,), jnp.int32),
        pltpu.VMEM((_TREES * 64,), jnp.float32),
        pltpu.VMEM((2, _SBLK, _TREES), jnp.float32),
        pltpu.SemaphoreType.DMA,
    ],
)
def _sc_traverse(xp_hbm, foci_hbm, leaf_hbm, out_hbm, xp_v, foci_v, leaf_v,
                 ob_v, sem0):
    wid = lax.axis_index("s") * 2 + lax.axis_index("c")
    base = wid * _ROWS_PER_W
    pltpu.sync_copy(foci_hbm, foci_v)
    pltpu.sync_copy(leaf_hbm, leaf_v)
    pltpu.sync_copy(xp_hbm.at[pl.ds(base * _WORDS, _ROWS_PER_W * _WORDS)], xp_v)

    lanes = jnp.arange(16, dtype=jnp.int32)
    n_blk = _ROWS_PER_W // _SBLK

    def sample_blk_compute(sb, buf):
        def tree_vec(tv, c2):
            t = tv * 16 + lanes
            lb = t * 64
            fbm = lb - t - 1
            f0 = plsc.load_gather(foci_v, [fbm + 1])
            w0 = f0 >> 4
            s0 = f0 & 15
            outs = []
            for s in range(_SBLK):
                xbase = (sb * _SBLK + s) * _WORDS
                wrd = plsc.load_gather(xp_v, [xbase + w0])
                n = 2 + ((lax.shift_right_logical(wrd, s0)) & 1)
                for _ in range(_DEPTH - 1):
                    f = plsc.load_gather(foci_v, [fbm + n])
                    wrd = plsc.load_gather(xp_v, [xbase + (f >> 4)])
                    dec = (lax.shift_right_logical(wrd, f & 15)) & 1
                    n = (n << 1) | dec
                outs.append(plsc.load_gather(leaf_v, [lb + n - 64]))
            for s in range(_SBLK):
                ob_v[buf, s, pl.ds(tv * 16, 16)] = outs[s]
            return c2

        lax.fori_loop(0, _TVECS, tree_vec, 0)

    def drain_one():
        pltpu.make_async_copy(
            ob_v.at[0],
            out_hbm.at[pl.ds(base, _SBLK)],
            sem0,
        ).wait()

    def sample_blk(sb, carry):
        buf = sb & 1
        sample_blk_compute(sb, buf)

        @pl.when(sb >= 1)
        def _wait_prev():
            drain_one()

        pltpu.async_copy(
            ob_v.at[buf],
            out_hbm.at[pl.ds(base + sb * _SBLK, _SBLK)],
            sem0,
        )
        return carry

    lax.fori_loop(0, n_blk, sample_blk, 0)
    drain_one()


def kernel(x, node_outputs, node_foci):
    xp = _pack_bits(x)
    foci_flat = node_foci.reshape(-1).astype(jnp.int32)
    leaf_flat = node_outputs[:, _LEAF0:, :].reshape(-1).astype(jnp.float32)
    return _sc_traverse(xp.reshape(-1), foci_flat, leaf_flat)


# speedup vs baseline: 1.1442x; 1.1442x over previous
import functools

import jax
import jax.numpy as jnp
from jax import lax
from jax.experimental import pallas as pl
from jax.experimental.pallas import tpu as pltpu
from jax.experimental.pallas import tpu_sc as plsc

_BATCH = 8192
_WIDTH = 1024
_TREES = 512
_DEPTH = 6
_NODES = 2 ** _DEPTH - 1
_LEAF0 = 2 ** (_DEPTH - 1) - 1
_WORDS = _WIDTH // 16

_NWORKERS = 32
_ROWS_PER_W = _BATCH // _NWORKERS
_SBLK = 16
_XROW = 128
_TVECS = _TREES // 16


def _pack_body(x_ref, p_ref, o_ref):
    acc = jnp.dot(x_ref[...], p_ref[...], preferred_element_type=jnp.float32)
    o_ref[...] = acc.astype(jnp.int32)


def _pack_bits(x):
    j = jnp.arange(_WIDTH, dtype=jnp.int32)
    c = jnp.arange(_XROW, dtype=jnp.int32)
    p = jnp.where((j[:, None] // 16) == c[None, :],
                  (2.0 ** (j % 16).astype(jnp.float32))[:, None],
                  0.0).astype(jnp.float32)
    return pl.pallas_call(
        _pack_body,
        grid=(16,),
        in_specs=[
            pl.BlockSpec((_BATCH // 16, _WIDTH), lambda i: (i, 0)),
            pl.BlockSpec((_WIDTH, _XROW), lambda i: (0, 0)),
        ],
        out_specs=pl.BlockSpec((_BATCH // 16, _XROW), lambda i: (i, 0)),
        out_shape=jax.ShapeDtypeStruct((_BATCH, _XROW), jnp.int32),
    )(x, p)


@functools.partial(
    pl.kernel,
    mesh=plsc.VectorSubcoreMesh(core_axis_name="c", subcore_axis_name="s"),
    out_type=jax.ShapeDtypeStruct((_BATCH, _TREES), jnp.float32),
    compiler_params=pltpu.CompilerParams(needs_layout_passes=False),
    scratch_types=[
        pltpu.VMEM((_ROWS_PER_W * _XROW,), jnp.int32),
        pltpu.VMEM((_TREES * _NODES,), jnp.int32),
        pltpu.VMEM((_TREES * 64,), jnp.float32),
        pltpu.VMEM((2, _SBLK, _TREES), jnp.float32),
        pltpu.SemaphoreType.DMA,
    ],
)
def _sc_traverse(xp_hbm, foci_hbm, leaf_hbm, out_hbm, xp_v, foci_v, leaf_v,
                 ob_v, sem0):
    wid = lax.axis_index("s") * 2 + lax.axis_index("c")
    base = wid * _ROWS_PER_W
    pltpu.sync_copy(foci_hbm, foci_v)
    pltpu.sync_copy(leaf_hbm, leaf_v)
    pltpu.sync_copy(xp_hbm.at[pl.ds(base * _XROW, _ROWS_PER_W * _XROW)], xp_v)

    lanes = jnp.arange(16, dtype=jnp.int32)
    n_blk = _ROWS_PER_W // _SBLK

    def sample_blk_compute(sb, buf):
        def tree_vec(tv, c2):
            t = tv * 16 + lanes
            lb = t * 64
            fbm = lb - t - 1
            f0 = plsc.load_gather(foci_v, [fbm + 1])
            w0 = f0 >> 4
            s0 = f0 & 15
            outs = []
            for s in range(_SBLK):
                xbase = (sb * _SBLK + s) * _XROW
                wrd = plsc.load_gather(xp_v, [xbase + w0])
                n = 2 + ((lax.shift_right_logical(wrd, s0)) & 1)
                for _ in range(_DEPTH - 1):
                    f = plsc.load_gather(foci_v, [fbm + n])
                    wrd = plsc.load_gather(xp_v, [xbase + (f >> 4)])
                    dec = (lax.shift_right_logical(wrd, f & 15)) & 1
                    n = (n << 1) | dec
                outs.append(plsc.load_gather(leaf_v, [lb + n - 64]))
            for s in range(_SBLK):
                ob_v[buf, s, pl.ds(tv * 16, 16)] = outs[s]
            return c2

        lax.fori_loop(0, _TVECS, tree_vec, 0)

    def drain_one():
        pltpu.make_async_copy(
            ob_v.at[0],
            out_hbm.at[pl.ds(base, _SBLK)],
            sem0,
        ).wait()

    def sample_blk(sb, carry):
        buf = sb & 1
        sample_blk_compute(sb, buf)

        @pl.when(sb >= 1)
        def _wait_prev():
            drain_one()

        pltpu.async_copy(
            ob_v.at[buf],
            out_hbm.at[pl.ds(base + sb * _SBLK, _SBLK)],
            sem0,
        )
        return carry

    lax.fori_loop(0, n_blk, sample_blk, 0)
    drain_one()


def kernel(x, node_outputs, node_foci):
    xp = _pack_bits(x)
    foci_flat = node_foci.reshape(-1).astype(jnp.int32)
    leaf_flat = node_outputs[:, _LEAF0:, :].reshape(-1).astype(jnp.float32)
    return _sc_traverse(xp.reshape(-1), foci_flat, leaf_flat)


# speedup vs baseline: 1.1996x; 1.0484x over previous
import functools

import jax
import jax.numpy as jnp
from jax import lax
from jax.experimental import pallas as pl
from jax.experimental.pallas import tpu as pltpu
from jax.experimental.pallas import tpu_sc as plsc

_BATCH = 8192
_WIDTH = 1024
_TREES = 512
_DEPTH = 6
_NODES = 2 ** _DEPTH - 1
_LEAF0 = 2 ** (_DEPTH - 1) - 1
_WORDS = _WIDTH // 16

_NWORKERS = 32
_ROWS_PER_W = _BATCH // _NWORKERS
_SBLK = 32
_XROW = 128
_TVECS = _TREES // 16


def _pack_body(x_ref, p_ref, o_ref):
    acc = jnp.dot(x_ref[...], p_ref[...], preferred_element_type=jnp.float32)
    o_ref[...] = acc.astype(jnp.int32)


def _pack_bits(x):
    j = jnp.arange(_WIDTH, dtype=jnp.int32)
    c = jnp.arange(_XROW, dtype=jnp.int32)
    p = jnp.where((j[:, None] // 16) == c[None, :],
                  (2.0 ** (j % 16).astype(jnp.float32))[:, None],
                  0.0).astype(jnp.float32)
    return pl.pallas_call(
        _pack_body,
        grid=(16,),
        in_specs=[
            pl.BlockSpec((_BATCH // 16, _WIDTH), lambda i: (i, 0)),
            pl.BlockSpec((_WIDTH, _XROW), lambda i: (0, 0)),
        ],
        out_specs=pl.BlockSpec((_BATCH // 16, _XROW), lambda i: (i, 0)),
        out_shape=jax.ShapeDtypeStruct((_BATCH, _XROW), jnp.int32),
    )(x, p)


@functools.partial(
    pl.kernel,
    mesh=plsc.VectorSubcoreMesh(core_axis_name="c", subcore_axis_name="s"),
    out_type=jax.ShapeDtypeStruct((_BATCH, _TREES), jnp.float32),
    compiler_params=pltpu.CompilerParams(needs_layout_passes=False),
    scratch_types=[
        pltpu.VMEM((_ROWS_PER_W * _XROW,), jnp.int32),
        pltpu.VMEM((_TREES * _NODES,), jnp.int32),
        pltpu.VMEM((_TREES * 64,), jnp.float32),
        pltpu.VMEM((2, _SBLK, _TREES), jnp.float32),
        pltpu.SemaphoreType.DMA,
    ],
)
def _sc_traverse(xp_hbm, foci_hbm, leaf_hbm, out_hbm, xp_v, foci_v, leaf_v,
                 ob_v, sem0):
    wid = lax.axis_index("s") * 2 + lax.axis_index("c")
    base = wid * _ROWS_PER_W
    pltpu.sync_copy(foci_hbm, foci_v)
    pltpu.sync_copy(leaf_hbm, leaf_v)
    pltpu.sync_copy(xp_hbm.at[pl.ds(base * _XROW, _ROWS_PER_W * _XROW)], xp_v)

    lanes = jnp.arange(16, dtype=jnp.int32)
    n_blk = _ROWS_PER_W // _SBLK

    def sample_blk_compute(sb, buf):
        def tree_vec(tv, c2):
            t = tv * 16 + lanes
            lb = t * 64
            fbm = lb - t - 1
            f0 = plsc.load_gather(foci_v, [fbm + 1])
            w0 = f0 >> 4
            s0 = f0 & 15
            outs = []
            for s in range(_SBLK):
                xbase = (sb * _SBLK + s) * _XROW
                wrd = plsc.load_gather(xp_v, [xbase + w0])
                n = 2 + ((lax.shift_right_logical(wrd, s0)) & 1)
                for _ in range(_DEPTH - 1):
                    f = plsc.load_gather(foci_v, [fbm + n])
                    wrd = plsc.load_gather(xp_v, [xbase + (f >> 4)])
                    dec = (lax.shift_right_logical(wrd, f & 15)) & 1
                    n = (n << 1) | dec
                outs.append(plsc.load_gather(leaf_v, [lb + n - 64]))
            for s in range(_SBLK):
                ob_v[buf, s, pl.ds(tv * 16, 16)] = outs[s]
            return c2

        lax.fori_loop(0, _TVECS, tree_vec, 0)

    def drain_one():
        pltpu.make_async_copy(
            ob_v.at[0],
            out_hbm.at[pl.ds(base, _SBLK)],
            sem0,
        ).wait()

    def sample_blk(sb, carry):
        buf = sb & 1
        sample_blk_compute(sb, buf)

        @pl.when(sb >= 1)
        def _wait_prev():
            drain_one()

        pltpu.async_copy(
            ob_v.at[buf],
            out_hbm.at[pl.ds(base + sb * _SBLK, _SBLK)],
            sem0,
        )
        return carry

    lax.fori_loop(0, n_blk, sample_blk, 0)
    drain_one()


def kernel(x, node_outputs, node_foci):
    xp = _pack_bits(x)
    foci_flat = node_foci.reshape(-1).astype(jnp.int32)
    leaf_flat = node_outputs[:, _LEAF0:, :].reshape(-1).astype(jnp.float32)
    return _sc_traverse(xp.reshape(-1), foci_flat, leaf_flat)


# speedup vs baseline: 1.2952x; 1.0797x over previous
import functools

import jax
import jax.numpy as jnp
from jax import lax
from jax.experimental import pallas as pl
from jax.experimental.pallas import tpu as pltpu
from jax.experimental.pallas import tpu_sc as plsc

_BATCH = 8192
_WIDTH = 1024
_TREES = 512
_DEPTH = 6
_NODES = 2 ** _DEPTH - 1
_LEAF0 = 2 ** (_DEPTH - 1) - 1
_WORDS = _WIDTH // 16

_NWORKERS = 32
_ROWS_PER_W = _BATCH // _NWORKERS
_SBLK = 32
_XROW = 128
_TVECS = _TREES // 16


def _pack_body(x_ref, p_ref, o_ref):
    acc = jnp.dot(x_ref[...], p_ref[...], preferred_element_type=jnp.float32)
    o_ref[...] = acc.astype(jnp.int32)


def _pack_bits(x):
    j = jnp.arange(_WIDTH, dtype=jnp.int32)
    c = jnp.arange(_XROW, dtype=jnp.int32)
    p = jnp.where((j[:, None] // 16) == c[None, :],
                  (2.0 ** (j % 16).astype(jnp.float32))[:, None],
                  0.0).astype(jnp.float32)
    return pl.pallas_call(
        _pack_body,
        grid=(16,),
        in_specs=[
            pl.BlockSpec((_BATCH // 16, _WIDTH), lambda i: (i, 0)),
            pl.BlockSpec((_WIDTH, _XROW), lambda i: (0, 0)),
        ],
        out_specs=pl.BlockSpec((_BATCH // 16, _XROW), lambda i: (i, 0)),
        out_shape=jax.ShapeDtypeStruct((_BATCH, _XROW), jnp.int32),
    )(x, p)


@functools.partial(
    pl.kernel,
    mesh=plsc.VectorSubcoreMesh(core_axis_name="c", subcore_axis_name="s"),
    out_type=jax.ShapeDtypeStruct((_BATCH, _TREES), jnp.float32),
    compiler_params=pltpu.CompilerParams(needs_layout_passes=False),
    scratch_types=[
        pltpu.VMEM((_ROWS_PER_W * _XROW,), jnp.int32),
        pltpu.VMEM((_TREES * _NODES,), jnp.int32),
        pltpu.VMEM((_TREES * 64,), jnp.float32),
        pltpu.VMEM((2, _SBLK, _TREES), jnp.float32),
        pltpu.SemaphoreType.DMA,
    ],
)
def _sc_traverse(xp_hbm, foci_hbm, leaf_hbm, out_hbm, xp_v, foci_v, leaf_v,
                 ob_v, sem0):
    wid = lax.axis_index("s") * 2 + lax.axis_index("c")
    base = wid * _ROWS_PER_W
    pltpu.sync_copy(foci_hbm, foci_v)
    pltpu.sync_copy(leaf_hbm, leaf_v)
    pltpu.sync_copy(xp_hbm.at[pl.ds(base * _XROW, _ROWS_PER_W * _XROW)], xp_v)

    lanes = jnp.arange(16, dtype=jnp.int32)
    n_blk = _ROWS_PER_W // _SBLK

    def sample_blk_compute(sb, buf):
        def tree_vec(tv, c2):
            t = tv * 16 + lanes
            lb = t * 64
            fbm = lb - t - 1
            f0 = plsc.load_gather(foci_v, [fbm + 1])
            w0 = f0 >> 4
            s0 = f0 & 15
            outs = []
            for s in range(_SBLK):
                xbase = (sb * _SBLK + s) * _XROW
                wrd = plsc.load_gather(xp_v, [xbase + w0])
                n = 2 + ((lax.shift_right_logical(wrd, s0)) & 1)
                for _ in range(_DEPTH - 1):
                    f = plsc.load_gather(foci_v, [fbm + n])
                    wrd = plsc.load_gather(xp_v, [xbase + (f >> 4)])
                    dec = (lax.shift_right_logical(wrd, f & 15)) & 1
                    n = (n << 1) | dec
                outs.append(plsc.load_gather(leaf_v, [lb + n - 64]))
            for s in range(_SBLK):
                ob_v[buf, s, pl.ds(tv * 16, 16)] = outs[s]
            return c2

        lax.fori_loop(0, _TVECS, tree_vec, 0)

    def drain_one():
        pltpu.make_async_copy(
            ob_v.at[0],
            out_hbm.at[pl.ds(base, _SBLK)],
            sem0,
        ).wait()

    def sample_blk(sb, carry):
        buf = sb & 1
        sample_blk_compute(sb, buf)

        @pl.when(sb >= 1)
        def _wait_prev():
            drain_one()

        pltpu.async_copy(
            ob_v.at[buf],
            out_hbm.at[pl.ds(base + sb * _SBLK, _SBLK)],
            sem0,
        )
        return carry

    lax.fori_loop(0, n_blk, sample_blk, 0)
    drain_one()


def kernel(x, node_outputs, node_foci):
    xp = _pack_bits(x)
    foci_flat = node_foci.reshape(-1).astype(jnp.int32)
    leaf_flat = (
        node_outputs.reshape(_TREES, _NODES * 2)[:, 2 * _LEAF0:]
        .reshape(-1).astype(jnp.float32)
    )
    return _sc_traverse(xp.reshape(-1), foci_flat, leaf_flat)
